# baseline (speedup 1.0000x reference)
import functools

import jax
import jax.numpy as jnp
from jax import lax
from jax.experimental import pallas as pl
from jax.experimental.pallas import tpu as pltpu
from jax.experimental.pallas import tpu_sc as plsc

N_FEAT = 128
N_GATES = 64
MAX_CONN = N_FEAT + 2 + N_GATES
PAD_CONN = 208
CHUNKS = PAD_CONN // 16
SUB = 8
LANES = 256
BLK = SUB * LANES


def _topk_sc_body(nc, gpw, gw_hbm, idx_hbm, row_v, prob_v, out_v):
    wid = lax.axis_index("s") * nc + lax.axis_index("c")
    big = jnp.int32(1 << 30)

    for j in range(gpw):
        g = wid * gpw + j
        n_valid = N_FEAT + 2 + g
        pltpu.sync_copy(gw_hbm.at[g], row_v)

        def chunk(c):
            x = row_v[pl.ds(c * 16, 16)]
            col = lax.iota(jnp.int32, 16) + c * 16
            return jnp.where(col < n_valid, x, -1e30), col

        def p_max(c, m):
            x, _ = chunk(c)
            return jnp.maximum(m, jnp.max(x))

        m = lax.fori_loop(0, CHUNKS, p_max, jnp.float32(-1e30))

        def p_expsum(c, s):
            x, col = chunk(c)
            e = jnp.where(col < n_valid, jnp.exp(x - m), 0.0)
            prob_v[pl.ds(c * 16, 16)] = e
            return s + jnp.sum(e)

        s = lax.fori_loop(0, CHUNKS, p_expsum, jnp.float32(0.0))

        def p_div(c, m1):
            p = prob_v[pl.ds(c * 16, 16)] / s
            prob_v[pl.ds(c * 16, 16)] = p
            return jnp.maximum(m1, jnp.max(p))

        m1 = lax.fori_loop(0, CHUNKS, p_div, jnp.float32(-1.0))

        def p_arg1(c, i1):
            p = prob_v[pl.ds(c * 16, 16)]
            col = lax.iota(jnp.int32, 16) + c * 16
            return jnp.minimum(i1, jnp.min(jnp.where(p == m1, col, big)))

        i1 = lax.fori_loop(0, CHUNKS, p_arg1, big)

        def p_max2(c, m2):
            p = prob_v[pl.ds(c * 16, 16)]
            col = lax.iota(jnp.int32, 16) + c * 16
            p = jnp.where(col == i1, -1.0, p)
            return jnp.maximum(m2, jnp.max(p))

        m2 = lax.fori_loop(0, CHUNKS, p_max2, jnp.float32(-1.0))

        def p_arg2(c, i2):
            p = prob_v[pl.ds(c * 16, 16)]
            col = lax.iota(jnp.int32, 16) + c * 16
            p = jnp.where(col == i1, -1.0, p)
            return jnp.minimum(i2, jnp.min(jnp.where(p == m2, col, big)))

        i2 = lax.fori_loop(0, CHUNKS, p_arg2, big)

        lane = lax.iota(jnp.int32, 16)
        out_v[...] = jnp.where(lane == 0, i1, jnp.where(lane == 1, i2, 0))
        pltpu.sync_copy(out_v, idx_hbm.at[g])


def _chain_kernel(idx_ref, x_ref, g_ref, avail_ref):
    avail_ref[0:N_FEAT] = x_ref[...]
    avail_ref[N_FEAT] = jnp.zeros((SUB, LANES), jnp.float32)
    avail_ref[N_FEAT + 1] = jnp.ones((SUB, LANES), jnp.float32)

    def step(g, carry):
        ia = idx_ref[g, 0]
        ib = idx_ref[g, 1]
        avail_ref[N_FEAT + 2 + g] = 1.0 - avail_ref[ia] * avail_ref[ib]
        return carry

    lax.fori_loop(0, N_GATES, step, 0)
    g_ref[...] = avail_ref[N_FEAT + 2:]


def _matmul_kernel(wt_ref, g_ref, scale_ref, out_ref):
    out_ref[...] = (
        jnp.dot(wt_ref[...], g_ref[...], preferred_element_type=jnp.float32)
        * scale_ref[...]
    )


def kernel(X, gate_weights, output_weights, output_scale):
    n = X.shape[0]
    n_out = output_weights.shape[1]
    nblk = n // BLK

    info = plsc.get_sparse_core_info()
    nc, ns = info.num_cores, info.num_subcores
    gpw = N_GATES // (nc * ns)

    gw_pad = jnp.pad(gate_weights, ((0, 0), (0, PAD_CONN - MAX_CONN)))
    topk = functools.partial(
        pl.kernel,
        mesh=plsc.VectorSubcoreMesh(core_axis_name="c", subcore_axis_name="s"),
        compiler_params=pltpu.CompilerParams(needs_layout_passes=False),
        out_type=jax.ShapeDtypeStruct((N_GATES, 16), jnp.int32),
        scratch_types=[
            pltpu.VMEM((PAD_CONN,), jnp.float32),
            pltpu.VMEM((PAD_CONN,), jnp.float32),
            pltpu.VMEM((16,), jnp.int32),
        ],
    )(functools.partial(_topk_sc_body, nc, gpw))
    idx = topk(gw_pad)[:, :2]

    xt = X.T.reshape(N_FEAT, n // LANES, LANES)
    g3 = pl.pallas_call(
        _chain_kernel,
        grid=(nblk,),
        in_specs=[
            pl.BlockSpec(memory_space=pltpu.SMEM),
            pl.BlockSpec((N_FEAT, SUB, LANES), lambda i: (0, i, 0)),
        ],
        out_specs=pl.BlockSpec((N_GATES, SUB, LANES), lambda i: (0, i, 0)),
        out_shape=jax.ShapeDtypeStruct((N_GATES, n // LANES, LANES), jnp.float32),
        scratch_shapes=[pltpu.VMEM((MAX_CONN, SUB, LANES), jnp.float32)],
    )(idx, xt)
    g2 = g3.reshape(N_GATES, n)

    outt = pl.pallas_call(
        _matmul_kernel,
        grid=(nblk,),
        in_specs=[
            pl.BlockSpec((n_out, N_GATES), lambda i: (0, 0)),
            pl.BlockSpec((N_GATES, BLK), lambda i: (0, i)),
            pl.BlockSpec((n_out, 1), lambda i: (0, 0)),
        ],
        out_specs=pl.BlockSpec((n_out, BLK), lambda i: (0, i)),
        out_shape=jax.ShapeDtypeStruct((n_out, n), jnp.float32),
    )(output_weights.T, g2, output_scale.reshape(n_out, 1))
    return outt.T


# speedup vs baseline: 1.4201x; 1.4201x over previous
import functools

import jax
import jax.numpy as jnp
from jax import lax
from jax.experimental import pallas as pl
from jax.experimental.pallas import tpu as pltpu
from jax.experimental.pallas import tpu_sc as plsc

N_FEAT = 128
N_GATES = 64
MAX_CONN = N_FEAT + 2 + N_GATES
PAD_CONN = 208
CHUNKS = PAD_CONN // 16
SUB = 8
LANES = 256
BLK = SUB * LANES


def _topk_sc_body(nc, gpw, gw_hbm, idx_hbm, row_v, prob_v, out_v):
    wid = lax.axis_index("s") * nc + lax.axis_index("c")
    big = jnp.int32(1 << 30)

    for j in range(gpw):
        g = wid * gpw + j
        n_valid = N_FEAT + 2 + g
        pltpu.sync_copy(gw_hbm.at[g], row_v)

        def chunk(c):
            x = row_v[pl.ds(c * 16, 16)]
            col = lax.iota(jnp.int32, 16) + c * 16
            return jnp.where(col < n_valid, x, -1e30), col

        def p_max(c, m):
            x, _ = chunk(c)
            return jnp.maximum(m, jnp.max(x))

        m = lax.fori_loop(0, CHUNKS, p_max, jnp.float32(-1e30))

        def p_expsum(c, s):
            x, col = chunk(c)
            e = jnp.where(col < n_valid, jnp.exp(x - m), 0.0)
            prob_v[pl.ds(c * 16, 16)] = e
            return s + jnp.sum(e)

        s = lax.fori_loop(0, CHUNKS, p_expsum, jnp.float32(0.0))

        def p_div(c, m1):
            p = prob_v[pl.ds(c * 16, 16)] / s
            prob_v[pl.ds(c * 16, 16)] = p
            return jnp.maximum(m1, jnp.max(p))

        m1 = lax.fori_loop(0, CHUNKS, p_div, jnp.float32(-1.0))

        def p_arg1(c, i1):
            p = prob_v[pl.ds(c * 16, 16)]
            col = lax.iota(jnp.int32, 16) + c * 16
            return jnp.minimum(i1, jnp.min(jnp.where(p == m1, col, big)))

        i1 = lax.fori_loop(0, CHUNKS, p_arg1, big)

        def p_max2(c, m2):
            p = prob_v[pl.ds(c * 16, 16)]
            col = lax.iota(jnp.int32, 16) + c * 16
            p = jnp.where(col == i1, -1.0, p)
            return jnp.maximum(m2, jnp.max(p))

        m2 = lax.fori_loop(0, CHUNKS, p_max2, jnp.float32(-1.0))

        def p_arg2(c, i2):
            p = prob_v[pl.ds(c * 16, 16)]
            col = lax.iota(jnp.int32, 16) + c * 16
            p = jnp.where(col == i1, -1.0, p)
            return jnp.minimum(i2, jnp.min(jnp.where(p == m2, col, big)))

        i2 = lax.fori_loop(0, CHUNKS, p_arg2, big)

        lane = lax.iota(jnp.int32, 16)
        out_v[...] = jnp.where(lane == 0, i1, jnp.where(lane == 1, i2, 0))
        pltpu.sync_copy(out_v, idx_hbm.at[g])


def _fused_chain_kernel(n_out, idx_ref, wt_ref, scale_ref, x_ref, out_ref,
                        avail_ref):
    for j in range(SUB):
        avail_ref[0:N_FEAT, j] = x_ref[j].T
    avail_ref[N_FEAT] = jnp.zeros((SUB, LANES), jnp.float32)
    avail_ref[N_FEAT + 1] = jnp.ones((SUB, LANES), jnp.float32)

    acc = [jnp.zeros((SUB, LANES), jnp.float32) for _ in range(n_out)]
    for g in range(N_GATES):
        ia = idx_ref[g, 0]
        ib = idx_ref[g, 1]
        row = 1.0 - avail_ref[ia] * avail_ref[ib]
        avail_ref[N_FEAT + 2 + g] = row
        for o in range(n_out):
            acc[o] = acc[o] + wt_ref[o, g] * row
    for o in range(n_out):
        out_ref[o] = acc[o] * scale_ref[o]


def kernel(X, gate_weights, output_weights, output_scale):
    n = X.shape[0]
    n_out = output_weights.shape[1]
    nblk = n // BLK

    info = plsc.get_sparse_core_info()
    nc, ns = info.num_cores, info.num_subcores
    gpw = N_GATES // (nc * ns)

    gw_pad = jnp.pad(gate_weights, ((0, 0), (0, PAD_CONN - MAX_CONN)))
    topk = functools.partial(
        pl.kernel,
        mesh=plsc.VectorSubcoreMesh(core_axis_name="c", subcore_axis_name="s"),
        compiler_params=pltpu.CompilerParams(needs_layout_passes=False),
        out_type=jax.ShapeDtypeStruct((N_GATES, 16), jnp.int32),
        scratch_types=[
            pltpu.VMEM((PAD_CONN,), jnp.float32),
            pltpu.VMEM((PAD_CONN,), jnp.float32),
            pltpu.VMEM((16,), jnp.int32),
        ],
    )(functools.partial(_topk_sc_body, nc, gpw))
    idx = topk(gw_pad)

    x3 = X.reshape(n // LANES, LANES, N_FEAT)
    out3 = pl.pallas_call(
        functools.partial(_fused_chain_kernel, n_out),
        grid=(nblk,),
        in_specs=[
            pl.BlockSpec(memory_space=pltpu.SMEM),
            pl.BlockSpec(memory_space=pltpu.SMEM),
            pl.BlockSpec(memory_space=pltpu.SMEM),
            pl.BlockSpec((SUB, LANES, N_FEAT), lambda i: (i, 0, 0)),
        ],
        out_specs=pl.BlockSpec((n_out, SUB, LANES), lambda i: (0, i, 0)),
        out_shape=jax.ShapeDtypeStruct((n_out, n // LANES, LANES), jnp.float32),
        scratch_shapes=[pltpu.VMEM((MAX_CONN, SUB, LANES), jnp.float32)],
    )(idx, output_weights.T, output_scale, x3)
    return out3.reshape(n_out, n).T
